# Initial kernel scaffold; baseline (speedup 1.0000x reference)
import jax
import jax.numpy as jnp
from jax.experimental import pallas as pl


def kernel(x, idx2vec):
    raise NotImplementedError("write your pallas kernel here")


# speedup vs baseline: 1.5047x; 1.5047x over previous
import functools

import jax
import jax.numpy as jnp
from jax import lax
from jax.experimental import pallas as pl
from jax.experimental.pallas import tpu as pltpu
from jax.experimental.pallas import tpu_sc as plsc

EMBED = 32
NC, NS = 2, 16
NW = NC * NS
CHUNK = 640
K = 4


@jax.jit
def _gather_sc(idx2, table):
    nw, tok_per_w = idx2.shape
    B = nw * tok_per_w
    n_chunks = tok_per_w // CHUNK
    n_iters = n_chunks // K
    assert n_chunks % K == 0
    mesh = plsc.VectorSubcoreMesh(core_axis_name="c", subcore_axis_name="s")

    @functools.partial(
        pl.kernel,
        out_type=jax.ShapeDtypeStruct((B, EMBED), jnp.float32),
        mesh=mesh,
        compiler_params=pltpu.CompilerParams(use_tc_tiling_on_sc=False),
        scratch_types=(
            [pltpu.VMEM((tok_per_w,), jnp.int32)]
            + [pltpu.VMEM((CHUNK, EMBED), jnp.float32) for _ in range(K)]
            + [pltpu.SemaphoreType.DMA for _ in range(2 * K)]
        ),
    )
    def k(idx_hbm, table_hbm, out_hbm, idx_v, *bufs_sems):
        bufs = bufs_sems[:K]
        gsem = bufs_sems[K:2 * K]
        wsem = bufs_sems[2 * K:]
        wid = lax.axis_index("s") * NC + lax.axis_index("c")
        base = wid * tok_per_w
        pltpu.sync_copy(idx_hbm.at[wid], idx_v)

        def fire_gather(b, c):
            return pltpu.async_copy(
                table_hbm.at[idx_v.at[pl.ds(c * CHUNK, CHUNK)]],
                bufs[b], gsem[b])

        for b in range(K):
            fire_gather(b, b)

        def body(i, carry):
            for b in range(K):
                c = i * K + b
                pltpu.make_async_copy(
                    table_hbm.at[idx_v.at[pl.ds(0, CHUNK)]],
                    bufs[b], gsem[b]).wait()
                wb = pltpu.async_copy(
                    bufs[b], out_hbm.at[pl.ds(base + c * CHUNK, CHUNK)],
                    wsem[b])
                wb.wait()

                @pl.when(c + K < n_chunks)
                def _():
                    fire_gather(b, c + K)
            return carry

        lax.fori_loop(0, n_iters, body, 0)

    return k(idx2, table)


def kernel(x, idx2vec):
    batch, length = x.shape
    B = batch * length
    idx2 = x.reshape(NW, B // NW)
    out = _gather_sc(idx2, idx2vec)
    return out.reshape(batch, length, EMBED)


# speedup vs baseline: 1.9593x; 1.3021x over previous
import functools

import jax
import jax.numpy as jnp
from jax import lax
from jax.experimental import pallas as pl
from jax.experimental.pallas import tpu as pltpu
from jax.experimental.pallas import tpu_sc as plsc

EMBED = 32
NC, NS = 2, 16
NW = NC * NS
BW = 128
K = 4


@jax.jit
def _gather_sc(xT, table):
    length, batch = xT.shape
    assert batch == NW * BW
    n_groups = length
    n_iters = n_groups // K
    assert n_groups % K == 0
    mesh = plsc.VectorSubcoreMesh(core_axis_name="c", subcore_axis_name="s")

    @functools.partial(
        pl.kernel,
        out_type=jax.ShapeDtypeStruct((batch, length * EMBED), jnp.float32),
        mesh=mesh,
        compiler_params=pltpu.CompilerParams(use_tc_tiling_on_sc=False),
        scratch_types=(
            [pltpu.VMEM((length, BW), jnp.int32)]
            + [pltpu.VMEM((BW, EMBED), jnp.float32) for _ in range(K)]
            + [pltpu.SemaphoreType.DMA for _ in range(2 * K)]
        ),
    )
    def k(xT_hbm, table_hbm, out_hbm, idx_v, *bufs_sems):
        bufs = bufs_sems[:K]
        gsem = bufs_sems[K:2 * K]
        wsem = bufs_sems[2 * K:]
        wid = lax.axis_index("s") * NC + lax.axis_index("c")
        b0 = wid * BW
        pltpu.sync_copy(xT_hbm.at[:, pl.ds(b0, BW)], idx_v)

        def fire_gather(b, t):
            return pltpu.async_copy(
                table_hbm.at[idx_v.at[t]], bufs[b], gsem[b])

        for b in range(K):
            fire_gather(b, b)

        def body(i, carry):
            for b in range(K):
                t = i * K + b
                pltpu.make_async_copy(
                    table_hbm.at[idx_v.at[0]], bufs[b], gsem[b]).wait()
                pltpu.async_copy(
                    bufs[b],
                    out_hbm.at[pl.ds(b0, BW), pl.ds(t * EMBED, EMBED)],
                    wsem[b]).wait()

                @pl.when(t + K < n_groups)
                def _():
                    fire_gather(b, t + K)
            return carry

        lax.fori_loop(0, n_iters, body, 0)

    return k(xT, table)


def kernel(x, idx2vec):
    batch, length = x.shape
    out2 = _gather_sc(x.T, idx2vec)
    return out2.reshape(batch, length, EMBED)
